# baseline (speedup 1.0000x reference)
import functools

import jax
import jax.numpy as jnp
from jax import lax
from jax.experimental import pallas as pl
from jax.experimental.pallas import tpu as pltpu
from jax.experimental.pallas import tpu_sc as plsc

NUM_EMB = 1000000
DIM = 32
BATCH = 4096
NUM_IDX = 50
TOTAL = BATCH * NUM_IDX

NC = 2
NS = 16
NW = NC * NS
B_PER_W = TOTAL // NW
CHUNK = 1600
N_CHUNKS = B_PER_W // CHUNK


@functools.partial(
    pl.kernel,
    mesh=plsc.VectorSubcoreMesh(core_axis_name="c", subcore_axis_name="s"),
    out_type=jax.ShapeDtypeStruct((TOTAL, DIM), jnp.float32),
    scratch_types=[
        pltpu.VMEM((CHUNK,), jnp.int32),
        pltpu.VMEM((CHUNK, DIM), jnp.float32),
        pltpu.SemaphoreType.DMA,
    ],
    compiler_params=pltpu.CompilerParams(use_tc_tiling_on_sc=False),
)
def _gather(table_hbm, idx_hbm, out_hbm, idx_v, rows_v, sem):
    wid = lax.axis_index("s") * NC + lax.axis_index("c")
    base = wid * B_PER_W
    for i in range(N_CHUNKS):
        off = base + i * CHUNK
        pltpu.sync_copy(idx_hbm.at[pl.ds(off, CHUNK)], idx_v)
        pltpu.async_copy(table_hbm.at[idx_v], rows_v, sem).wait()
        pltpu.sync_copy(rows_v, out_hbm.at[pl.ds(off, CHUNK)])


def kernel(index, weight):
    idx_flat = index.reshape(TOTAL).astype(jnp.int32)
    out = _gather(weight, idx_flat)
    return out.reshape(BATCH, NUM_IDX, DIM)


# speedup vs baseline: 1.0051x; 1.0051x over previous
import functools

import jax
import jax.numpy as jnp
from jax import lax
from jax.experimental import pallas as pl
from jax.experimental.pallas import tpu as pltpu
from jax.experimental.pallas import tpu_sc as plsc

NUM_EMB = 1000000
DIM = 32
BATCH = 4096
NUM_IDX = 50
TOTAL = BATCH * NUM_IDX

NC = 2
NS = 16
NW = NC * NS
B_PER_W = TOTAL // NW
CHUNK = 800
N_CHUNKS = B_PER_W // CHUNK
NBUF = 4


@functools.partial(
    pl.kernel,
    mesh=plsc.VectorSubcoreMesh(core_axis_name="c", subcore_axis_name="s"),
    out_type=jax.ShapeDtypeStruct((TOTAL, DIM), jnp.float32),
    scratch_types=[
        pltpu.VMEM((N_CHUNKS, CHUNK), jnp.int32),
        pltpu.VMEM((NBUF, CHUNK, DIM), jnp.float32),
    ] + [pltpu.SemaphoreType.DMA] * NBUF,
    compiler_params=pltpu.CompilerParams(use_tc_tiling_on_sc=False),
)
def _gather(table_hbm, idx_hbm, out_hbm, idx_v, rows_v, *sems):
    wid = lax.axis_index("s") * NC + lax.axis_index("c")
    base = wid * B_PER_W
    pltpu.sync_copy(idx_hbm.at[wid], idx_v)
    cps = [None] * N_CHUNKS
    for i in range(NBUF):
        cps[i] = pltpu.async_copy(
            table_hbm.at[idx_v.at[i]], rows_v.at[i % NBUF], sems[i % NBUF])
    for i in range(N_CHUNKS):
        cps[i].wait()
        pltpu.sync_copy(rows_v.at[i % NBUF],
                        out_hbm.at[pl.ds(base + i * CHUNK, CHUNK)])
        nxt = i + NBUF
        if nxt < N_CHUNKS:
            cps[nxt] = pltpu.async_copy(
                table_hbm.at[idx_v.at[nxt]], rows_v.at[nxt % NBUF],
                sems[nxt % NBUF])


def kernel(index, weight):
    idx = index.reshape(NW, N_CHUNKS, CHUNK).astype(jnp.int32)
    out = _gather(weight, idx)
    return out.reshape(BATCH, NUM_IDX, DIM)


# speedup vs baseline: 1.2213x; 1.2151x over previous
import functools

import jax
import jax.numpy as jnp
from jax import lax
from jax.experimental import pallas as pl
from jax.experimental.pallas import tpu as pltpu
from jax.experimental.pallas import tpu_sc as plsc

NUM_EMB = 1000000
DIM = 32
BATCH = 4096
NUM_IDX = 50

NC = 2
NS = 16
NW = NC * NS
ROWS_PER_W = BATCH // NW
GROUP = 16
N_GROUPS = ROWS_PER_W // GROUP
NBUF = 2


@functools.partial(
    pl.kernel,
    mesh=plsc.VectorSubcoreMesh(core_axis_name="c", subcore_axis_name="s"),
    out_type=jax.ShapeDtypeStruct((BATCH, NUM_IDX, DIM), jnp.float32),
    scratch_types=[
        pltpu.VMEM((ROWS_PER_W, NUM_IDX), jnp.int32),
        pltpu.VMEM((NBUF, GROUP, NUM_IDX, DIM), jnp.float32),
    ] + [pltpu.SemaphoreType.DMA] * (2 * NBUF),
    compiler_params=pltpu.CompilerParams(use_tc_tiling_on_sc=False),
)
def _gather(table_hbm, idx_hbm, out_hbm, idx_v, rows_v, *sems):
    gsems, wsems = sems[:NBUF], sems[NBUF:]
    wid = lax.axis_index("s") * NC + lax.axis_index("c")
    row0 = wid * ROWS_PER_W
    pltpu.sync_copy(idx_hbm.at[pl.ds(row0, ROWS_PER_W), :], idx_v)
    wr = [None] * N_GROUPS
    for g in range(N_GROUPS):
        b = g % NBUF
        if g >= NBUF:
            wr[g - NBUF].wait()
        cps = [
            pltpu.async_copy(
                table_hbm.at[idx_v.at[g * GROUP + j]], rows_v.at[b, j],
                gsems[b])
            for j in range(GROUP)
        ]
        for cp in cps:
            cp.wait()
        wr[g] = pltpu.async_copy(
            rows_v.at[b], out_hbm.at[pl.ds(row0 + g * GROUP, GROUP)],
            wsems[b])
    for g in range(N_GROUPS - NBUF, N_GROUPS):
        wr[g].wait()


def kernel(index, weight):
    return _gather(weight, index.astype(jnp.int32))
